# baseline (speedup 1.0000x reference)
import functools

import jax
import jax.numpy as jnp
from jax import lax
from jax.experimental import pallas as pl
from jax.experimental.pallas import tpu as pltpu
from jax.experimental.pallas import tpu_sc as plsc

NF = 6
D = 8
XD = NF * D
H = 192
LANES = 128
CB = 64

_SQRT_HALF = 0.7071067811865476


def _detile_body(in_ref, out_ref):
    for cb in range(CB):
        out_ref[cb] = in_ref[:, cb * LANES:(cb + 1) * LANES]


def _detile(tt):
    v = tt.shape[1]
    nblk = -(-v // LANES)
    nb = -(-nblk // CB)
    out = pl.pallas_call(
        _detile_body,
        grid=(nb,),
        in_specs=[pl.BlockSpec((D, LANES * CB), lambda c: (0, c))],
        out_specs=pl.BlockSpec((CB, D, LANES), lambda c: (c, 0, 0)),
        out_shape=jax.ShapeDtypeStruct((nb * CB, D, LANES), jnp.float32),
    )(tt)
    return out.reshape(-1)


def _gather_body(i0, i1, i2, i3, i4, i5, t0, t1, t2, t3, t4, t5,
                 o0, o1, o2, o3, o4, o5,
                 x0, x1, x2, x3, x4, x5, g0, g1, g2, g3, g4, g5,
                 r0, r1, r2, r3, r4, r5, sem,
                 *, chunk, nc):
    wid = lax.axis_index("s") * nc + lax.axis_index("c")
    base = wid * chunk
    irefs = (i0, i1, i2, i3, i4, i5)
    trefs = (t0, t1, t2, t3, t4, t5)
    orefs = (o0, o1, o2, o3, o4, o5)
    idxs = (x0, x1, x2, x3, x4, x5)
    fidxs = (g0, g1, g2, g3, g4, g5)
    rows = (r0, r1, r2, r3, r4, r5)
    for f in range(NF):
        pltpu.sync_copy(irefs[f].at[pl.ds(base, chunk)], idxs[f])
    for f in range(NF):
        for j in range(chunk // 16):
            i = idxs[f][pl.ds(j * 16, 16)]
            b = (lax.shift_left(lax.shift_right_logical(i, 7), 10)
                 | (i & 127))
            for d in range(D):
                fidxs[f][pl.ds(d * chunk + j * 16, 16)] = b + d * LANES
    copies = [pltpu.async_copy(trefs[f].at[fidxs[f]], rows[f], sem)
              for f in range(NF)]
    for c in copies:
        c.wait()
    for f in range(NF):
        pltpu.sync_copy(rows[f], orefs[f].at[wid])


def _make_gather(batch):
    info = plsc.get_sparse_core_info()
    nc, ns = info.num_cores, info.num_subcores
    nw = nc * ns
    chunk = batch // nw
    mesh = plsc.VectorSubcoreMesh(core_axis_name="c", subcore_axis_name="s")
    return pl.kernel(
        functools.partial(_gather_body, chunk=chunk, nc=nc),
        mesh=mesh,
        compiler_params=pltpu.CompilerParams(use_tc_tiling_on_sc=False),
        out_type=[jax.ShapeDtypeStruct((nw, D * chunk), jnp.float32)
                  for _ in range(NF)],
        scratch_types=(
            [pltpu.VMEM((chunk,), jnp.int32) for _ in range(NF)]
            + [pltpu.VMEM((D * chunk,), jnp.int32) for _ in range(NF)]
            + [pltpu.VMEM((D * chunk,), jnp.float32) for _ in range(NF)]
            + [pltpu.SemaphoreType.DMA]
        ),
    )


def _gelu(z):
    return 0.5 * z * (1.0 + lax.erf(z * _SQRT_HALF))


def _dense_body(e0, e1, e2, e3, e4, e5,
                wkT, bk2, wqT, bq2, wvT, bv2,
                wwT, bw2, wd0T, wd1T, bd2, wlT, bl2, out):
    xt = jnp.concatenate(
        [e0[...], e1[...], e2[...], e3[...], e4[...], e5[...]], axis=0)
    t = xt.shape[1]

    def dot_nn(w, v):
        return lax.dot_general(w, v, (((1,), (0,)), ((), ())),
                               preferred_element_type=jnp.float32)

    kt = _gelu(dot_nn(wkT[...], xt) + bk2[...])
    qt = _gelu(dot_nn(wqT[...], xt) + bq2[...])
    vt = _gelu(dot_nn(wvT[...], xt) + bv2[...])

    qb = [qt[g * D:(g + 1) * D, :] for g in range(NF)]
    yb = []
    for e in range(D):
        acc = None
        for g in range(NF):
            krow = jnp.broadcast_to(kt[g * D + e:g * D + e + 1, :], (D, t))
            term = krow * qb[g]
            acc = term if acc is None else acc + term
        yb.append(acc)
    res_rows = []
    for f in range(NF):
        acc = None
        for e in range(D):
            vrow = jnp.broadcast_to(vt[f * D + e:f * D + e + 1, :], (D, t))
            term = vrow * yb[e]
            acc = term if acc is None else acc + term
        res_rows.append(acc)
    x0t = jnp.concatenate(res_rows, axis=0) + xt

    x1 = jnp.maximum(dot_nn(wwT[...], xt) + bw2[...], 0.0)
    z = dot_nn(wd0T[...], x0t) + dot_nn(wd1T[...], x1) + bd2[...]
    h = jnp.maximum(z, 0.0)
    out[...] = dot_nn(wlT[...], h) + bl2[...]


def _dense(embs, wkT, bk2, wqT, bq2, wvT, bv2, wwT, bw2,
           wd0T, wd1T, bd2, wlT, bl2, batch, t):
    grid = (batch // t,)
    emb_spec = pl.BlockSpec((D, t), lambda i: (0, i))
    fixed = lambda shape: pl.BlockSpec(shape, lambda i: (0, 0))
    return pl.pallas_call(
        _dense_body,
        grid=grid,
        in_specs=[emb_spec] * NF + [
            fixed((XD, XD)), fixed((XD, 1)),
            fixed((XD, XD)), fixed((XD, 1)),
            fixed((XD, XD)), fixed((XD, 1)),
            fixed((H, XD)), fixed((H, 1)),
            fixed((H, XD)), fixed((H, H)), fixed((H, 1)),
            fixed((1, H)), fixed((1, 1)),
        ],
        out_specs=pl.BlockSpec((1, t), lambda i: (0, i)),
        out_shape=jax.ShapeDtypeStruct((1, batch), jnp.float32),
    )(*embs, wkT, bk2, wqT, bq2, wvT, bv2, wwT, bw2,
      wd0T, wd1T, bd2, wlT, bl2)


def kernel(movie_id, user_id, user_zip_code, user_occupation_text,
           user_gender, bucketized_user_age,
           table_movie_id, table_user_id, table_user_zip_code,
           table_user_occupation_text, table_user_gender,
           table_bucketized_user_age,
           Wk, bk, Wq, bq, Wv, bv, Ww, bw, Wd, bd, Wl, bl):
    batch = movie_id.shape[0]
    flats = [_detile(t.T) for t in (
        table_movie_id, table_user_id, table_user_zip_code,
        table_user_occupation_text, table_user_gender,
        table_bucketized_user_age)]
    idx_all = (movie_id, user_id, user_zip_code, user_occupation_text,
               user_gender, bucketized_user_age)
    nhalf = 2
    hb = batch // nhalf
    gather = _make_gather(hb)
    parts = []
    for h in range(nhalf):
        raw = gather(*[i[h * hb:(h + 1) * hb] for i in idx_all], *flats)
        nw = raw[0].shape[0]
        chunk = hb // nw
        embs = [r.reshape(nw, D, chunk).transpose(1, 0, 2).reshape(D, hb)
                for r in raw]
        parts.append(_dense(
            embs,
            Wk.T, bk.reshape(XD, 1), Wq.T, bq.reshape(XD, 1),
            Wv.T, bv.reshape(XD, 1),
            Ww.T, bw.reshape(H, 1),
            Wd[:XD, :].T, Wd[XD:, :].T, bd.reshape(H, 1),
            Wl.T, bl.reshape(1, 1),
            hb, 512))
    return jnp.concatenate(parts, axis=1).reshape(batch, 1)


# speedup vs baseline: 1.2569x; 1.2569x over previous
import functools

import jax
import jax.numpy as jnp
from jax import lax
from jax.experimental import pallas as pl
from jax.experimental.pallas import tpu as pltpu
from jax.experimental.pallas import tpu_sc as plsc

NF = 6
D = 8
XD = NF * D
H = 192
LANES = 128
CB = 512

_SQRT_HALF = 0.7071067811865476


def _detile_body(in_ref, out_ref):
    for cb in range(CB):
        out_ref[cb] = in_ref[:, cb * LANES:(cb + 1) * LANES]


def _detile(tt):
    v = tt.shape[1]
    nblk = -(-v // LANES)
    nb = -(-nblk // CB)
    out = pl.pallas_call(
        _detile_body,
        grid=(nb,),
        in_specs=[pl.BlockSpec((D, LANES * CB), lambda c: (0, c))],
        out_specs=pl.BlockSpec((CB, D, LANES), lambda c: (c, 0, 0)),
        out_shape=jax.ShapeDtypeStruct((nb * CB, D, LANES), jnp.float32),
    )(tt)
    return out.reshape(-1)


def _gather_body(i0, i1, i2, i3, i4, i5, t0, t1, t2, t3, t4, t5,
                 o0, o1, o2, o3, o4, o5,
                 x0, x1, x2, x3, x4, x5, g0, g1, g2, g3, g4, g5,
                 r0, r1, r2, r3, r4, r5, sem,
                 *, chunk, nc):
    wid = lax.axis_index("s") * nc + lax.axis_index("c")
    base = wid * chunk
    irefs = (i0, i1, i2, i3, i4, i5)
    trefs = (t0, t1, t2, t3, t4, t5)
    orefs = (o0, o1, o2, o3, o4, o5)
    idxs = (x0, x1, x2, x3, x4, x5)
    fidxs = (g0, g1, g2, g3, g4, g5)
    rows = (r0, r1, r2, r3, r4, r5)
    for f in range(NF):
        pltpu.sync_copy(irefs[f].at[pl.ds(base, chunk)], idxs[f])
    for f in range(NF):
        for j in range(chunk // 16):
            i = idxs[f][pl.ds(j * 16, 16)]
            b = (lax.shift_left(lax.shift_right_logical(i, 7), 10)
                 | (i & 127))
            for d in range(D):
                fidxs[f][pl.ds(d * chunk + j * 16, 16)] = b + d * LANES
    copies = [pltpu.async_copy(trefs[f].at[fidxs[f]], rows[f], sem)
              for f in range(NF)]
    for c in copies:
        c.wait()
    for f in range(NF):
        pltpu.sync_copy(rows[f], orefs[f].at[wid])


def _make_gather(batch):
    info = plsc.get_sparse_core_info()
    nc, ns = info.num_cores, info.num_subcores
    nw = nc * ns
    chunk = batch // nw
    mesh = plsc.VectorSubcoreMesh(core_axis_name="c", subcore_axis_name="s")
    return pl.kernel(
        functools.partial(_gather_body, chunk=chunk, nc=nc),
        mesh=mesh,
        compiler_params=pltpu.CompilerParams(use_tc_tiling_on_sc=False),
        out_type=[jax.ShapeDtypeStruct((nw, D * chunk), jnp.float32)
                  for _ in range(NF)],
        scratch_types=(
            [pltpu.VMEM((chunk,), jnp.int32) for _ in range(NF)]
            + [pltpu.VMEM((D * chunk,), jnp.int32) for _ in range(NF)]
            + [pltpu.VMEM((D * chunk,), jnp.float32) for _ in range(NF)]
            + [pltpu.SemaphoreType.DMA]
        ),
    )


def _gelu(z):
    return 0.5 * z * (1.0 + lax.erf(z * _SQRT_HALF))


def _dense_body(e0, e1, e2, e3, e4, e5,
                wkT, bk2, wqT, bq2, wvT, bv2,
                wwT, bw2, wd0T, wd1T, bd2, wlT, bl2, out):
    xt = jnp.concatenate(
        [e0[...], e1[...], e2[...], e3[...], e4[...], e5[...]], axis=0)
    t = xt.shape[1]

    def dot_nn(w, v):
        return lax.dot_general(w, v, (((1,), (0,)), ((), ())),
                               preferred_element_type=jnp.float32)

    kt = _gelu(dot_nn(wkT[...], xt) + bk2[...])
    qt = _gelu(dot_nn(wqT[...], xt) + bq2[...])
    vt = _gelu(dot_nn(wvT[...], xt) + bv2[...])

    qb = [qt[g * D:(g + 1) * D, :] for g in range(NF)]
    yb = []
    for e in range(D):
        acc = None
        for g in range(NF):
            krow = jnp.broadcast_to(kt[g * D + e:g * D + e + 1, :], (D, t))
            term = krow * qb[g]
            acc = term if acc is None else acc + term
        yb.append(acc)
    res_rows = []
    for f in range(NF):
        acc = None
        for e in range(D):
            vrow = jnp.broadcast_to(vt[f * D + e:f * D + e + 1, :], (D, t))
            term = vrow * yb[e]
            acc = term if acc is None else acc + term
        res_rows.append(acc)
    x0t = jnp.concatenate(res_rows, axis=0) + xt

    x1 = jnp.maximum(dot_nn(wwT[...], xt) + bw2[...], 0.0)
    z = dot_nn(wd0T[...], x0t) + dot_nn(wd1T[...], x1) + bd2[...]
    h = jnp.maximum(z, 0.0)
    out[...] = dot_nn(wlT[...], h) + bl2[...]


def _dense(embs, wkT, bk2, wqT, bq2, wvT, bv2, wwT, bw2,
           wd0T, wd1T, bd2, wlT, bl2, batch, t):
    grid = (batch // t,)
    emb_spec = pl.BlockSpec((D, t), lambda i: (0, i))
    fixed = lambda shape: pl.BlockSpec(shape, lambda i: (0, 0))
    return pl.pallas_call(
        _dense_body,
        grid=grid,
        in_specs=[emb_spec] * NF + [
            fixed((XD, XD)), fixed((XD, 1)),
            fixed((XD, XD)), fixed((XD, 1)),
            fixed((XD, XD)), fixed((XD, 1)),
            fixed((H, XD)), fixed((H, 1)),
            fixed((H, XD)), fixed((H, H)), fixed((H, 1)),
            fixed((1, H)), fixed((1, 1)),
        ],
        out_specs=pl.BlockSpec((1, t), lambda i: (0, i)),
        out_shape=jax.ShapeDtypeStruct((1, batch), jnp.float32),
    )(*embs, wkT, bk2, wqT, bq2, wvT, bv2, wwT, bw2,
      wd0T, wd1T, bd2, wlT, bl2)


def kernel(movie_id, user_id, user_zip_code, user_occupation_text,
           user_gender, bucketized_user_age,
           table_movie_id, table_user_id, table_user_zip_code,
           table_user_occupation_text, table_user_gender,
           table_bucketized_user_age,
           Wk, bk, Wq, bq, Wv, bv, Ww, bw, Wd, bd, Wl, bl):
    batch = movie_id.shape[0]
    flats = [_detile(t.T) for t in (
        table_movie_id, table_user_id, table_user_zip_code,
        table_user_occupation_text, table_user_gender,
        table_bucketized_user_age)]
    raw = _make_gather(batch)(
        movie_id, user_id, user_zip_code, user_occupation_text,
        user_gender, bucketized_user_age, *flats)
    nw = raw[0].shape[0]
    chunk = batch // nw
    embs = [r.reshape(nw, D, chunk).transpose(1, 0, 2).reshape(D, batch)
            for r in raw]
    scores = _dense(
        embs,
        Wk.T, bk.reshape(XD, 1), Wq.T, bq.reshape(XD, 1),
        Wv.T, bv.reshape(XD, 1),
        Ww.T, bw.reshape(H, 1),
        Wd[:XD, :].T, Wd[XD:, :].T, bd.reshape(H, 1),
        Wl.T, bl.reshape(1, 1),
        batch, 512)
    return scores.reshape(batch, 1)


# speedup vs baseline: 1.9856x; 1.5798x over previous
import functools

import jax
import jax.numpy as jnp
from jax import lax
from jax.experimental import pallas as pl
from jax.experimental.pallas import tpu as pltpu
from jax.experimental.pallas import tpu_sc as plsc

NF = 6
D = 8
XD = NF * D
H = 192
LANES = 128
CB = 512

_SQRT_HALF = 0.7071067811865476


def _detile_body(in_ref, out_ref, *, cb):
    for c in range(cb):
        out_ref[c] = in_ref[:, c * LANES:(c + 1) * LANES]


def _detile(tt):
    v = tt.shape[1]
    nblk = -(-v // LANES)
    cb = min(CB, nblk)
    nb = -(-nblk // cb)
    out = pl.pallas_call(
        functools.partial(_detile_body, cb=cb),
        grid=(nb,),
        in_specs=[pl.BlockSpec((D, LANES * cb), lambda c: (0, c))],
        out_specs=pl.BlockSpec((cb, D, LANES), lambda c: (c, 0, 0)),
        out_shape=jax.ShapeDtypeStruct((nb * cb, D, LANES), jnp.float32),
    )(tt)
    return out.reshape(-1)


_STAGED = (0, 2, 3, 4, 5)
_USER = 1


def _gather_body(i0, i1, i2, i3, i4, i5, t0, t1, t2, t3, t4, t5,
                 o0, o1, o2, o3, o4, o5,
                 x0, x1, x2, x3, x4, x5, g0, g1, g2, g3, g4, g5,
                 r0, r1, r2, r3, r4, r5, s0, s2, s3, s4, s5, sem,
                 *, chunk, nc):
    sid = lax.axis_index("s")
    wid = sid * nc + lax.axis_index("c")
    base = wid * chunk
    irefs = (i0, i1, i2, i3, i4, i5)
    trefs = (t0, t1, t2, t3, t4, t5)
    orefs = (o0, o1, o2, o3, o4, o5)
    idxs = (x0, x1, x2, x3, x4, x5)
    fidxs = (g0, g1, g2, g3, g4, g5)
    rows = (r0, r1, r2, r3, r4, r5)
    shared = {0: s0, 2: s2, 3: s3, 4: s4, 5: s5}
    for f in range(NF):
        pltpu.sync_copy(irefs[f].at[pl.ds(base, chunk)], idxs[f])
    for f in range(NF):
        for j in range(chunk // 16):
            i = idxs[f][pl.ds(j * 16, 16)]
            b = (lax.shift_left(lax.shift_right_logical(i, 7), 10)
                 | (i & 127))
            for d in range(D):
                fidxs[f][pl.ds(d * chunk + j * 16, 16)] = b + d * LANES
    copies = [pltpu.async_copy(trefs[_USER].at[fidxs[_USER]], rows[_USER],
                               sem)]
    @pl.when(sid == 0)
    def _stage():
        for f in _STAGED:
            pltpu.sync_copy(trefs[f], shared[f])
    plsc.subcore_barrier()
    for f in _STAGED:
        copies.append(pltpu.async_copy(shared[f].at[fidxs[f]], rows[f], sem))
    for c in copies:
        c.wait()
    for f in range(NF):
        pltpu.sync_copy(rows[f], orefs[f].at[wid])


def _make_gather(batch, flens):
    info = plsc.get_sparse_core_info()
    nc, ns = info.num_cores, info.num_subcores
    nw = nc * ns
    chunk = batch // nw
    mesh = plsc.VectorSubcoreMesh(core_axis_name="c", subcore_axis_name="s")
    return pl.kernel(
        functools.partial(_gather_body, chunk=chunk, nc=nc),
        mesh=mesh,
        compiler_params=pltpu.CompilerParams(use_tc_tiling_on_sc=False),
        out_type=[jax.ShapeDtypeStruct((nw, D * chunk), jnp.float32)
                  for _ in range(NF)],
        scratch_types=(
            [pltpu.VMEM((chunk,), jnp.int32) for _ in range(NF)]
            + [pltpu.VMEM((D * chunk,), jnp.int32) for _ in range(NF)]
            + [pltpu.VMEM((D * chunk,), jnp.float32) for _ in range(NF)]
            + [pltpu.VMEM_SHARED((flens[f],), jnp.float32)
               for f in _STAGED]
            + [pltpu.SemaphoreType.DMA]
        ),
    )


def _gelu(z):
    return 0.5 * z * (1.0 + lax.erf(z * _SQRT_HALF))


def _dense_body(e0, e1, e2, e3, e4, e5,
                wkT, bk2, wqT, bq2, wvT, bv2,
                wwT, bw2, wd0T, wd1T, bd2, wlT, bl2, out):
    xt = jnp.concatenate(
        [e0[...], e1[...], e2[...], e3[...], e4[...], e5[...]], axis=0)
    t = xt.shape[1]

    def dot_nn(w, v):
        return lax.dot_general(w, v, (((1,), (0,)), ((), ())),
                               preferred_element_type=jnp.float32)

    kt = _gelu(dot_nn(wkT[...], xt) + bk2[...])
    qt = _gelu(dot_nn(wqT[...], xt) + bq2[...])
    vt = _gelu(dot_nn(wvT[...], xt) + bv2[...])

    qb = [qt[g * D:(g + 1) * D, :] for g in range(NF)]
    yb = []
    for e in range(D):
        acc = None
        for g in range(NF):
            krow = jnp.broadcast_to(kt[g * D + e:g * D + e + 1, :], (D, t))
            term = krow * qb[g]
            acc = term if acc is None else acc + term
        yb.append(acc)
    res_rows = []
    for f in range(NF):
        acc = None
        for e in range(D):
            vrow = jnp.broadcast_to(vt[f * D + e:f * D + e + 1, :], (D, t))
            term = vrow * yb[e]
            acc = term if acc is None else acc + term
        res_rows.append(acc)
    x0t = jnp.concatenate(res_rows, axis=0) + xt

    x1 = jnp.maximum(dot_nn(wwT[...], xt) + bw2[...], 0.0)
    z = dot_nn(wd0T[...], x0t) + dot_nn(wd1T[...], x1) + bd2[...]
    h = jnp.maximum(z, 0.0)
    out[...] = dot_nn(wlT[...], h) + bl2[...]


def _dense(embs, wkT, bk2, wqT, bq2, wvT, bv2, wwT, bw2,
           wd0T, wd1T, bd2, wlT, bl2, batch, t):
    grid = (batch // t,)
    emb_spec = pl.BlockSpec((D, t), lambda i: (0, i))
    fixed = lambda shape: pl.BlockSpec(shape, lambda i: (0, 0))
    return pl.pallas_call(
        _dense_body,
        grid=grid,
        in_specs=[emb_spec] * NF + [
            fixed((XD, XD)), fixed((XD, 1)),
            fixed((XD, XD)), fixed((XD, 1)),
            fixed((XD, XD)), fixed((XD, 1)),
            fixed((H, XD)), fixed((H, 1)),
            fixed((H, XD)), fixed((H, H)), fixed((H, 1)),
            fixed((1, H)), fixed((1, 1)),
        ],
        out_specs=pl.BlockSpec((1, t), lambda i: (0, i)),
        out_shape=jax.ShapeDtypeStruct((1, batch), jnp.float32),
    )(*embs, wkT, bk2, wqT, bq2, wvT, bv2, wwT, bw2,
      wd0T, wd1T, bd2, wlT, bl2)


def kernel(movie_id, user_id, user_zip_code, user_occupation_text,
           user_gender, bucketized_user_age,
           table_movie_id, table_user_id, table_user_zip_code,
           table_user_occupation_text, table_user_gender,
           table_bucketized_user_age,
           Wk, bk, Wq, bq, Wv, bv, Ww, bw, Wd, bd, Wl, bl):
    batch = movie_id.shape[0]
    flats = [_detile(t.T) for t in (
        table_movie_id, table_user_id, table_user_zip_code,
        table_user_occupation_text, table_user_gender,
        table_bucketized_user_age)]
    raw = _make_gather(batch, [f.shape[0] for f in flats])(
        movie_id, user_id, user_zip_code, user_occupation_text,
        user_gender, bucketized_user_age, *flats)
    nw = raw[0].shape[0]
    chunk = batch // nw
    embs = [r.reshape(nw, D, chunk).transpose(1, 0, 2).reshape(D, batch)
            for r in raw]
    scores = _dense(
        embs,
        Wk.T, bk.reshape(XD, 1), Wq.T, bq.reshape(XD, 1),
        Wv.T, bv.reshape(XD, 1),
        Ww.T, bw.reshape(H, 1),
        Wd[:XD, :].T, Wd[XD:, :].T, bd.reshape(H, 1),
        Wl.T, bl.reshape(1, 1),
        batch, 512)
    return scores.reshape(batch, 1)
